# baseline (speedup 1.0000x reference)
import functools

import jax
import jax.numpy as jnp
from jax.experimental import pallas as pl
from jax.experimental.pallas import tpu as pltpu
from jax.experimental.pallas import tpu_sc as plsc

_E = 64
_K = 8
_BT = 1024
_NW = 32
_L = 16


def _logits_kernel(x_ref, w_ref, logits_ref):
    logits_ref[...] = jax.lax.dot_general(
        w_ref[...], x_ref[...], (((1,), (1,)), ((), ())),
        preferred_element_type=jnp.float32,
    )


def _sc_topk_kernel(logits_hbm, idx_hbm, score_hbm, lv, oi, ov):
    tokens_per_w = lv.shape[1]
    wid = jax.lax.axis_index("s") * 2 + jax.lax.axis_index("c")
    base = wid * tokens_per_w
    pltpu.sync_copy(logits_hbm.at[:, pl.ds(base, tokens_per_w)], lv)

    def group_body(g, _):
        off = g * _L
        neg = jnp.full((_L,), -1e30, dtype=jnp.float32)
        vals = [neg] * _K
        idxs = [jnp.full((_L,), _E, dtype=jnp.int32)] * _K
        for e in range(_E):
            cur_v = lv[e, pl.ds(off, _L)]
            cur_i = jnp.full((_L,), e, dtype=jnp.int32)
            for j in range(_K):
                take = cur_v > vals[j]
                nv = jnp.where(take, cur_v, vals[j])
                ni = jnp.where(take, cur_i, idxs[j])
                cur_v = jnp.where(take, vals[j], cur_v)
                cur_i = jnp.where(take, idxs[j], cur_i)
                vals[j] = nv
                idxs[j] = ni
        m = vals[0]
        es = [jnp.exp(v - m) for v in vals]
        den = es[0]
        for j in range(1, _K):
            den = den + es[j]
        for j in range(_K):
            oi[j, pl.ds(off, _L)] = idxs[j]
            ov[j, pl.ds(off, _L)] = es[j] / den
        return _

    jax.lax.fori_loop(0, tokens_per_w // _L, group_body, None)
    pltpu.sync_copy(oi, idx_hbm.at[:, pl.ds(base, tokens_per_w)])
    pltpu.sync_copy(ov, score_hbm.at[:, pl.ds(base, tokens_per_w)])


@functools.partial(jax.jit, static_argnames=())
def kernel(hidden_states, weight):
    b, s, d = hidden_states.shape
    t = b * s
    x = hidden_states.reshape(t, d)
    grid = (t // _BT,)
    logits = pl.pallas_call(
        _logits_kernel,
        grid=grid,
        in_specs=[
            pl.BlockSpec((_BT, d), lambda i: (i, 0)),
            pl.BlockSpec((_E, d), lambda i: (0, 0)),
        ],
        out_specs=pl.BlockSpec((_E, _BT), lambda i: (0, i)),
        out_shape=jax.ShapeDtypeStruct((_E, t), jnp.float32),
    )(x, weight)

    tokens_per_w = t // _NW
    mesh = plsc.VectorSubcoreMesh(
        core_axis_name="c", subcore_axis_name="s", num_cores=2, num_subcores=16
    )
    idx_t, scores_t = pl.kernel(
        _sc_topk_kernel,
        out_type=[
            jax.ShapeDtypeStruct((_K, t), jnp.int32),
            jax.ShapeDtypeStruct((_K, t), jnp.float32),
        ],
        mesh=mesh,
        scratch_types=[
            pltpu.VMEM((_E, tokens_per_w), jnp.float32),
            pltpu.VMEM((_K, tokens_per_w), jnp.int32),
            pltpu.VMEM((_K, tokens_per_w), jnp.float32),
        ],
    )(logits)
    aux_loss = jnp.zeros((), dtype=jnp.float32)
    return (idx_t.T, scores_t.T, aux_loss)
